# baseline (speedup 1.0000x reference)
import functools

import jax
import jax.numpy as jnp
from jax import lax
from jax.experimental import pallas as pl
from jax.experimental.pallas import tpu as pltpu
from jax.experimental.pallas import tpu_sc as plsc

CASCADE = 4
GRID = 128
M = GRID ** 3
N = M // 4 * 2
DECAY = 0.95
DENSITY_SCALE = 1.0

NC = 2
NS = 16
NW = NC * NS
SLICE = M // NW
EPT = N // NW

CHUNK = 2048
NGRP = EPT // CHUNK
K4WIN = 4096

PAD_MAX = CASCADE * NW * NW * 15
STAG_TRASH = CASCADE * N + PAD_MAX
STAGN = STAG_TRASH + NW * 16
SENTINEL = 0x7FFFFFF0

RANK_IS_ZERO_BASED = False


def _rank0(cnt):
    return cnt if RANK_IS_ZERO_BASED else cnt - 1


def _iota16():
    return lax.iota(jnp.int32, 16)


def _wid():
    return lax.axis_index("s") * NC + lax.axis_index("c")


_MESH = dict(
    mesh=plsc.VectorSubcoreMesh(core_axis_name="c", subcore_axis_name="s"),
    compiler_params=pltpu.CompilerParams(needs_layout_passes=False),
)


def _k1_body(idx_hbm, cnt_hbm, cnt_v, ib):
    wid = _wid()
    for u in range(8):
        cnt_v[pl.ds(u * 16, 16)] = jnp.zeros((16,), jnp.int32)
    for c in range(CASCADE):
        elem0 = c * N + wid * EPT

        def _win(w, carry):
            pltpu.sync_copy(idx_hbm.at[pl.ds(elem0 + w * K4WIN, K4WIN)], ib)

            def _vec(v, c2):
                iv = ib[pl.ds(v * 16, 16)]
                o = lax.shift_right_logical(iv, 16)
                cnt, last = plsc.scan_count(o)
                plsc.addupdate_scatter(
                    cnt_v, [o + (c * NW)], _rank0(cnt) + 1, mask=last)
                return c2
            lax.fori_loop(0, K4WIN // 16, _vec, 0)
            return carry
        lax.fori_loop(0, EPT // K4WIN, _win, 0)
    pltpu.sync_copy(cnt_v, cnt_hbm.at[pl.ds(wid * CASCADE * NW, CASCADE * NW)])


def _k2_body(cnt_hbm, offs_hbm, segb_hbm, cnt4k, offs4k, segb_v):
    wid = _wid()

    @pl.when(wid == 0)
    def _():
        pltpu.sync_copy(cnt_hbm, cnt4k)
        ii = _iota16()

        def _sb(p, carry):
            q = p * 16 + ii
            t = q & 31
            o = lax.shift_right_logical(q, 5) & 31
            c = lax.shift_right_logical(q, 10)
            src = t * (CASCADE * NW) + c * NW + o
            v = plsc.load_gather(cnt4k, [src])
            pv = (v + 15) & jnp.int32(-16)
            inc = plsc.cumsum(pv)
            ex = inc - pv + carry
            plsc.store_scatter(offs4k, [src], ex)
            plsc.store_scatter(segb_v, [c * NW + o], ex, mask=(t == 0))
            return carry + jnp.max(inc)
        total = lax.fori_loop(0, (CASCADE * NW * NW) // 16, _sb, jnp.int32(0))
        segb_v[pl.ds(CASCADE * NW, 16)] = jnp.full((16,), 1, jnp.int32) * total
        pltpu.sync_copy(offs4k, offs_hbm)
        pltpu.sync_copy(segb_v, segb_hbm)


def _k3_body(idx_hbm, sig_hbm, offs_hbm, sidx_hbm, ssig_hbm,
             offs_v, cur_v, ib, sb, dbuf0, dbuf1, vbuf0, vbuf1,
             wbuf0, wbuf1, dpad, vpad, wpad, sem_i, sem_s):
    dbufs, vbufs, wbufs = (dbuf0, dbuf1), (vbuf0, vbuf1), (wbuf0, wbuf1)
    wid = _wid()
    trash = jnp.int32(STAG_TRASH) + wid * 16
    pltpu.sync_copy(offs_hbm.at[pl.ds(wid * CASCADE * NW, CASCADE * NW)],
                    offs_v)
    ii = _iota16()

    for c in range(CASCADE):
        cur_v[pl.ds(0, 16)] = offs_v[pl.ds(c * NW, 16)]
        cur_v[pl.ds(16, 16)] = offs_v[pl.ds(c * NW + 16, 16)]
        elem0 = c * N + wid * EPT
        cbase = jnp.int32(c * M)

        def _group(g, ph):
            goff = elem0 + g * CHUNK
            pltpu.sync_copy(idx_hbm.at[pl.ds(goff, CHUNK)], ib)
            pltpu.sync_copy(sig_hbm.at[pl.ds(goff, CHUNK)], sb)

            @pl.when(g >= 2)
            def _():
                pltpu.make_async_copy(
                    vbufs[ph], sidx_hbm.at[pl.ds(0, CHUNK)],
                    sem_i.at[ph]).wait()
                pltpu.make_async_copy(
                    wbufs[ph], ssig_hbm.at[pl.ds(0, CHUNK)],
                    sem_s.at[ph]).wait()

            def _vec(v, c2):
                iv = ib[pl.ds(v * 16, 16)]
                sv = sb[pl.ds(v * 16, 16)]
                o = lax.shift_right_logical(iv, 16)
                cnt, last = plsc.scan_count(o)
                r0 = _rank0(cnt)
                base = plsc.load_gather(cur_v, [o])
                dest = base + r0
                plsc.store_scatter(cur_v, [o], dest + 1, mask=last)
                dbufs[ph][pl.ds(v * 16, 16)] = dest
                vbufs[ph][pl.ds(v * 16, 16)] = iv + cbase
                wbufs[ph][pl.ds(v * 16, 16)] = sv
                return c2
            lax.fori_loop(0, CHUNK // 16, _vec, 0)
            pltpu.async_copy(vbufs[ph], sidx_hbm.at[dbufs[ph]],
                             sem_i.at[ph])
            pltpu.async_copy(wbufs[ph], ssig_hbm.at[dbufs[ph]],
                             sem_s.at[ph])

        def _gpair(gp, carry):
            _group(gp * 2, 0)
            _group(gp * 2 + 1, 1)
            return carry
        lax.fori_loop(0, NGRP // 2, _gpair, 0)

        for ph in range(2):
            pltpu.make_async_copy(vbufs[ph], sidx_hbm.at[pl.ds(0, CHUNK)],
                                  sem_i.at[ph]).wait()
            pltpu.make_async_copy(wbufs[ph], ssig_hbm.at[pl.ds(0, CHUNK)],
                                  sem_s.at[ph]).wait()

        cv0 = cur_v[pl.ds(0, 16)]
        cv1 = cur_v[pl.ds(16, 16)]
        for o in range(NW):
            cur_o = cv0[o] if o < 16 else cv1[o - 16]
            npad = (-cur_o) & 15
            dd = jnp.where(ii < npad, cur_o + ii, trash + ii)
            dpad[pl.ds(o * 16, 16)] = dd
            vpad[pl.ds(o * 16, 16)] = jnp.full((16,), SENTINEL, jnp.int32)
            wpad[pl.ds(o * 16, 16)] = jnp.zeros((16,), jnp.float32)
        pltpu.sync_copy(vpad, sidx_hbm.at[dpad])
        pltpu.sync_copy(wpad, ssig_hbm.at[dpad])


def _k4_body(sidx_hbm, ssig_hbm, segb_hbm, tmp_hbm, segb_v, tmp_v, ib, sb):
    wid = _wid()
    ii = _iota16()
    pltpu.sync_copy(segb_hbm, segb_v)
    total = segb_v[pl.ds(CASCADE * NW, 16)][0]

    for c in range(CASCADE):
        se = plsc.load_gather(segb_v, [c * NW + wid + ii])
        start = se[0]
        end0 = se[1]
        base_flat = jnp.int32(c * M) + wid * SLICE

        def _ms(i, carry):
            for u in range(8):
                tmp_v[pl.ds((i * 8 + u) * 16, 16)] = jnp.full(
                    (16,), -1.0, jnp.float32)
            return carry
        lax.fori_loop(0, SLICE // 128, _ms, 0)

        nwin = lax.shift_right_logical(end0 - start + (K4WIN - 1), 12)

        def _win(w, carry):
            off_l = start + w * K4WIN
            off = pl.multiple_of(jnp.minimum(off_l, total - K4WIN), 16)
            pltpu.sync_copy(sidx_hbm.at[pl.ds(off, K4WIN)], ib)
            pltpu.sync_copy(ssig_hbm.at[pl.ds(off, K4WIN)], sb)

            def _vec(v, c2):
                iv = ib[pl.ds(v * 16, 16)]
                sv = sb[pl.ds(v * 16, 16)]
                gpos = off + v * 16 + ii
                loc = iv - base_flat
                ok = ((gpos >= off_l) & (gpos < end0)
                      & (loc >= 0) & (loc < SLICE))
                locc = jnp.where(ok, loc, 0)
                _, last = plsc.scan_count(locc, mask=ok)
                plsc.store_scatter(tmp_v, [locc], sv, mask=last)
                return c2
            lax.fori_loop(0, K4WIN // 16, _vec, 0)
            return carry
        lax.fori_loop(0, nwin, _win, 0)
        pltpu.sync_copy(tmp_v, tmp_hbm.at[pl.ds(c * M + wid * SLICE, SLICE)])


@jax.jit
def _sc_scatter(idx_flat, sig_flat):
    i32 = jnp.int32
    f32 = jnp.float32
    k1 = functools.partial(
        pl.kernel,
        out_type=jax.ShapeDtypeStruct((NW * CASCADE * NW,), i32),
        scratch_types=[
            pltpu.VMEM((CASCADE * NW,), i32),
            pltpu.VMEM((K4WIN,), i32),
        ],
        **_MESH,
    )(_k1_body)
    cnts = k1(idx_flat)

    k2 = functools.partial(
        pl.kernel,
        out_type=(jax.ShapeDtypeStruct((NW * CASCADE * NW,), i32),
                  jax.ShapeDtypeStruct((256,), i32)),
        scratch_types=[
            pltpu.VMEM((NW * CASCADE * NW,), i32),
            pltpu.VMEM((NW * CASCADE * NW,), i32),
            pltpu.VMEM((256,), i32),
        ],
        **_MESH,
    )(_k2_body)
    offs, segb = k2(cnts)

    k3 = functools.partial(
        pl.kernel,
        out_type=(jax.ShapeDtypeStruct((STAGN,), i32),
                  jax.ShapeDtypeStruct((STAGN,), f32)),
        scratch_types=[
            pltpu.VMEM((CASCADE * NW,), i32),
            pltpu.VMEM((NW,), i32),
            pltpu.VMEM((CHUNK,), i32),
            pltpu.VMEM((CHUNK,), f32),
            pltpu.VMEM((CHUNK,), i32),
            pltpu.VMEM((CHUNK,), i32),
            pltpu.VMEM((CHUNK,), i32),
            pltpu.VMEM((CHUNK,), i32),
            pltpu.VMEM((CHUNK,), f32),
            pltpu.VMEM((CHUNK,), f32),
            pltpu.VMEM((NW * 16,), i32),
            pltpu.VMEM((NW * 16,), i32),
            pltpu.VMEM((NW * 16,), f32),
            pltpu.SemaphoreType.DMA((2,)),
            pltpu.SemaphoreType.DMA((2,)),
        ],
        **_MESH,
    )(_k3_body)
    sidx, ssig = k3(idx_flat, sig_flat, offs)

    k4 = functools.partial(
        pl.kernel,
        out_type=jax.ShapeDtypeStruct((CASCADE * M,), f32),
        scratch_types=[
            pltpu.VMEM((256,), i32),
            pltpu.VMEM((SLICE,), f32),
            pltpu.VMEM((K4WIN,), i32),
            pltpu.VMEM((K4WIN,), f32),
        ],
        **_MESH,
    )(_k4_body)
    return k4(sidx, ssig, segb)


def _tc_body(g_ref, t_ref, o_ref):
    g = g_ref[...]
    t = t_ref[...]
    o_ref[...] = jnp.where((g >= 0.0) & (t >= 0.0),
                           jnp.maximum(g * DECAY, t), g)


@jax.jit
def _tc_combine(density_grid, tmp_grid):
    C, Mc = density_grid.shape
    rows, cols = 8192, C * Mc // 8192
    blk = 512
    out = pl.pallas_call(
        _tc_body,
        out_shape=jax.ShapeDtypeStruct((rows, cols), jnp.float32),
        grid=(rows // blk,),
        in_specs=[
            pl.BlockSpec((blk, cols), lambda i: (i, 0)),
            pl.BlockSpec((blk, cols), lambda i: (i, 0)),
        ],
        out_specs=pl.BlockSpec((blk, cols), lambda i: (i, 0)),
    )(density_grid.reshape(rows, cols), tmp_grid.reshape(rows, cols))
    return out.reshape(C, Mc)


def kernel(density_grid, sigmas, indices):
    C, Mc = density_grid.shape
    idx_flat = indices.reshape(-1)
    sig_flat = (sigmas * DENSITY_SCALE).reshape(-1)
    tmp = _sc_scatter(idx_flat, sig_flat).reshape(C, Mc)
    return _tc_combine(density_grid, tmp)


# speedup vs baseline: 7.9712x; 7.9712x over previous
import functools

import jax
import jax.numpy as jnp
from jax import lax
from jax.experimental import pallas as pl
from jax.experimental.pallas import tpu as pltpu
from jax.experimental.pallas import tpu_sc as plsc

CASCADE = 4
GRID = 128
M = GRID ** 3
N = M // 4 * 2
DECAY = 0.95
DENSITY_SCALE = 1.0

NC = 2
NS = 16
SLICE = M // (NC * NS)
NR = 4
QUART = N // NR
EPT = QUART // NS

CHUNK = 2048
NGRP = EPT // CHUNK
DWIN = 4096

CAPE = QUART + 15 * 256
CAPS = CAPE + NS * 16
SENTINEL = 0x7FFFFFF0


def _iota16():
    return lax.iota(jnp.int32, 16)


_MESH = dict(
    mesh=plsc.VectorSubcoreMesh(core_axis_name="c", subcore_axis_name="s"),
    compiler_params=pltpu.CompilerParams(needs_layout_passes=False),
)


def _k1_body(idx_hbm, cnt_hbm, cnt_v, ib):
    k = lax.axis_index("c")
    s = lax.axis_index("s")
    w2 = k * NS + s
    for u in range(16):
        cnt_v[pl.ds(u * 16, 16)] = jnp.zeros((16,), jnp.int32)
    for c in range(CASCADE):
        for r in range(NR):
            elem0 = c * N + r * QUART + s * EPT

            def _win(w, carry):
                pltpu.sync_copy(idx_hbm.at[pl.ds(elem0 + w * DWIN, DWIN)], ib)

                def _vec(v, c2):
                    iv = ib[pl.ds(v * 16, 16)]
                    o = lax.shift_right_logical(iv, 16)
                    keep = lax.shift_right_logical(o, 4) == k
                    o16 = o & 15
                    cnt, last = plsc.scan_count(o16, mask=keep)
                    plsc.addupdate_scatter(
                        cnt_v, [o16 + (c * 64 + r * 16)], cnt, mask=last)
                    return c2
                lax.fori_loop(0, DWIN // 16, _vec, 0)
                return carry
            lax.fori_loop(0, EPT // DWIN, _win, 0)
    pltpu.sync_copy(cnt_v, cnt_hbm.at[pl.ds(w2 * 256, 256)])


def _k2_body(cnt_hbm, offs_hbm, segb_hbm, cnt4k, offs4k, segb_v):
    k = lax.axis_index("c")
    s = lax.axis_index("s")

    @pl.when((k == 0) & (s == 0))
    def _():
        pltpu.sync_copy(cnt_hbm, cnt4k)
        ii = _iota16()

        def _sb(p, carry):
            q = p * 16 + ii
            qk = lax.shift_right_logical(q, 12)
            qc = lax.shift_right_logical(q, 10) & 3
            qr = lax.shift_right_logical(q, 8) & 3
            qo = lax.shift_right_logical(q, 4) & 15
            qs = q & 15
            src = (qk * NS + qs) * 256 + qc * 64 + qr * 16 + qo
            v = plsc.load_gather(cnt4k, [src])
            pv = (v + 15) & jnp.int32(-16)
            inc = plsc.cumsum(pv)
            ex = inc - pv + carry
            plsc.store_scatter(offs4k, [src], ex)
            grp = lax.shift_right_logical(p, 4)
            plsc.store_scatter(segb_v, [grp * 32 + qo], ex, mask=(qs == 0))
            carry_new = carry + jnp.max(inc)
            gpos = p & 15
            plsc.store_scatter(
                segb_v, [jnp.full((16,), 1, jnp.int32) * (grp * 32 + 16)],
                jnp.full((16,), 1, jnp.int32) * carry_new,
                mask=(ii == 0) & (gpos == 15))
            return jnp.where(gpos == 15, jnp.int32(0), carry_new)
        lax.fori_loop(0, 512, _sb, jnp.int32(0))
        pltpu.sync_copy(offs4k, offs_hbm)
        pltpu.sync_copy(segb_v, segb_hbm)


def _k34_body(idx_hbm, sig_hbm, offs_hbm, segb_hbm, tmp_hbm,
              offs_v, segb_v, cur_v, ib, sb,
              dbuf0, dbuf1, vbuf0, vbuf1, wbuf0, wbuf1,
              dpad, vpad, wpad, ib4, sb4, tmp_v,
              sidx_sp, ssig_sp, sem_i, sem_s):
    dbufs, vbufs, wbufs = (dbuf0, dbuf1), (vbuf0, vbuf1), (wbuf0, wbuf1)
    k = lax.axis_index("c")
    s = lax.axis_index("s")
    w2 = k * NS + s
    ii = _iota16()
    trash = jnp.int32(CAPE) + s * 16
    pltpu.sync_copy(offs_hbm.at[pl.ds(w2 * 256, 256)], offs_v)
    pltpu.sync_copy(segb_hbm, segb_v)
    own_base = (k * NS + s) * SLICE

    for c in range(CASCADE):
        for r in range(NR):
            cur_v[pl.ds(0, 16)] = offs_v[pl.ds(c * 64 + r * 16, 16)]
            elem0 = c * N + r * QUART + s * EPT

            def _group(g, ph):
                pltpu.sync_copy(idx_hbm.at[pl.ds(elem0 + g * CHUNK, CHUNK)],
                                ib)
                pltpu.sync_copy(sig_hbm.at[pl.ds(elem0 + g * CHUNK, CHUNK)],
                                sb)

                @pl.when(g >= 2)
                def _():
                    pltpu.make_async_copy(
                        vbufs[ph], sidx_sp.at[pl.ds(0, CHUNK)],
                        sem_i.at[ph]).wait()
                    pltpu.make_async_copy(
                        wbufs[ph], ssig_sp.at[pl.ds(0, CHUNK)],
                        sem_s.at[ph]).wait()

                def _vec(v, c2):
                    iv = ib[pl.ds(v * 16, 16)]
                    sv = sb[pl.ds(v * 16, 16)]
                    o = lax.shift_right_logical(iv, 16)
                    keep = lax.shift_right_logical(o, 4) == k
                    o16 = o & 15
                    cnt, last = plsc.scan_count(o16, mask=keep)
                    base = plsc.load_gather(cur_v, [o16])
                    dest = base + cnt - 1
                    plsc.store_scatter(cur_v, [o16], dest + 1, mask=last)
                    dbufs[ph][pl.ds(v * 16, 16)] = jnp.where(
                        keep, dest, trash + ii)
                    vbufs[ph][pl.ds(v * 16, 16)] = iv
                    wbufs[ph][pl.ds(v * 16, 16)] = sv
                    return c2
                lax.fori_loop(0, CHUNK // 16, _vec, 0)
                pltpu.async_copy(vbufs[ph], sidx_sp.at[dbufs[ph]],
                                 sem_i.at[ph])
                pltpu.async_copy(wbufs[ph], ssig_sp.at[dbufs[ph]],
                                 sem_s.at[ph])

            def _gpair(gp, carry):
                _group(gp * 2, 0)
                _group(gp * 2 + 1, 1)
                return carry
            lax.fori_loop(0, NGRP // 2, _gpair, 0)
            for ph in range(2):
                pltpu.make_async_copy(vbufs[ph], sidx_sp.at[pl.ds(0, CHUNK)],
                                      sem_i.at[ph]).wait()
                pltpu.make_async_copy(wbufs[ph], ssig_sp.at[pl.ds(0, CHUNK)],
                                      sem_s.at[ph]).wait()

            cv0 = cur_v[pl.ds(0, 16)]
            for o in range(NS):
                cur_o = cv0[o]
                npad = (-cur_o) & 15
                dd = jnp.where(ii < npad, cur_o + ii, trash + ii)
                dpad[pl.ds(o * 16, 16)] = dd
                vpad[pl.ds(o * 16, 16)] = jnp.full((16,), SENTINEL, jnp.int32)
                wpad[pl.ds(o * 16, 16)] = jnp.zeros((16,), jnp.float32)
            pltpu.sync_copy(vpad, sidx_sp.at[dpad])
            pltpu.sync_copy(wpad, ssig_sp.at[dpad])

            plsc.subcore_barrier()

            if r == 0:
                def _ms(i, carry):
                    for u in range(8):
                        tmp_v[pl.ds((i * 8 + u) * 16, 16)] = jnp.full(
                            (16,), -1.0, jnp.float32)
                    return carry
                lax.fori_loop(0, SLICE // 128, _ms, 0)

            grp = k * 16 + c * NR + r
            se = plsc.load_gather(segb_v, [grp * 32 + s + ii])
            start = se[0]
            end0 = se[1]
            nwin = lax.shift_right_logical(end0 - start + (DWIN - 1), 12)

            def _win(w, carry):
                off_l = start + w * DWIN
                off = pl.multiple_of(
                    jnp.maximum(jnp.minimum(off_l, jnp.int32(CAPE - DWIN)),
                                jnp.int32(0)), 16)
                pltpu.sync_copy(sidx_sp.at[pl.ds(off, DWIN)], ib4)
                pltpu.sync_copy(ssig_sp.at[pl.ds(off, DWIN)], sb4)

                def _vec(v, c2):
                    iv = ib4[pl.ds(v * 16, 16)]
                    sv = sb4[pl.ds(v * 16, 16)]
                    gpos = off + v * 16 + ii
                    loc = iv - own_base
                    ok = ((gpos >= off_l) & (gpos < end0)
                          & (loc >= 0) & (loc < SLICE))
                    locc = jnp.where(ok, loc, 0)
                    _, last = plsc.scan_count(locc, mask=ok)
                    plsc.store_scatter(tmp_v, [locc], sv, mask=last)
                    return c2
                lax.fori_loop(0, DWIN // 16, _vec, 0)
                return carry
            lax.fori_loop(0, nwin, _win, 0)

            if r == NR - 1:
                pltpu.sync_copy(
                    tmp_v, tmp_hbm.at[pl.ds(c * M + own_base, SLICE)])

            plsc.subcore_barrier()


@jax.jit
def _sc_scatter(idx_flat, sig_flat):
    i32 = jnp.int32
    f32 = jnp.float32
    k1 = functools.partial(
        pl.kernel,
        out_type=jax.ShapeDtypeStruct((NC * NS * 256,), i32),
        scratch_types=[
            pltpu.VMEM((256,), i32),
            pltpu.VMEM((DWIN,), i32),
        ],
        **_MESH,
    )(_k1_body)
    cnts = k1(idx_flat)

    k2 = functools.partial(
        pl.kernel,
        out_type=(jax.ShapeDtypeStruct((NC * NS * 256,), i32),
                  jax.ShapeDtypeStruct((1024,), i32)),
        scratch_types=[
            pltpu.VMEM((NC * NS * 256,), i32),
            pltpu.VMEM((NC * NS * 256,), i32),
            pltpu.VMEM((1024,), i32),
        ],
        **_MESH,
    )(_k2_body)
    offs, segb = k2(cnts)

    k34 = functools.partial(
        pl.kernel,
        out_type=jax.ShapeDtypeStruct((CASCADE * M,), f32),
        scratch_types=[
            pltpu.VMEM((256,), i32),
            pltpu.VMEM((1024,), i32),
            pltpu.VMEM((16,), i32),
            pltpu.VMEM((CHUNK,), i32),
            pltpu.VMEM((CHUNK,), f32),
            pltpu.VMEM((CHUNK,), i32),
            pltpu.VMEM((CHUNK,), i32),
            pltpu.VMEM((CHUNK,), i32),
            pltpu.VMEM((CHUNK,), i32),
            pltpu.VMEM((CHUNK,), f32),
            pltpu.VMEM((CHUNK,), f32),
            pltpu.VMEM((NS * 16,), i32),
            pltpu.VMEM((NS * 16,), i32),
            pltpu.VMEM((NS * 16,), f32),
            pltpu.VMEM((DWIN,), i32),
            pltpu.VMEM((DWIN,), f32),
            pltpu.VMEM((SLICE,), f32),
            pltpu.VMEM_SHARED((CAPS,), i32),
            pltpu.VMEM_SHARED((CAPS,), f32),
            pltpu.SemaphoreType.DMA((2,)),
            pltpu.SemaphoreType.DMA((2,)),
        ],
        **_MESH,
    )(_k34_body)
    return k34(idx_flat, sig_flat, offs, segb)


def _tc_body(g_ref, t_ref, o_ref):
    g = g_ref[...]
    t = t_ref[...]
    o_ref[...] = jnp.where((g >= 0.0) & (t >= 0.0),
                           jnp.maximum(g * DECAY, t), g)


@jax.jit
def _tc_combine(density_grid, tmp_grid):
    C, Mc = density_grid.shape
    rows, cols = 8192, C * Mc // 8192
    blk = 512
    out = pl.pallas_call(
        _tc_body,
        out_shape=jax.ShapeDtypeStruct((rows, cols), jnp.float32),
        grid=(rows // blk,),
        in_specs=[
            pl.BlockSpec((blk, cols), lambda i: (i, 0)),
            pl.BlockSpec((blk, cols), lambda i: (i, 0)),
        ],
        out_specs=pl.BlockSpec((blk, cols), lambda i: (i, 0)),
    )(density_grid.reshape(rows, cols), tmp_grid.reshape(rows, cols))
    return out.reshape(C, Mc)


def kernel(density_grid, sigmas, indices):
    C, Mc = density_grid.shape
    idx_flat = indices.reshape(-1)
    sig_flat = (sigmas * DENSITY_SCALE).reshape(-1)
    tmp = _sc_scatter(idx_flat, sig_flat).reshape(C, Mc)
    return _tc_combine(density_grid, tmp)


# speedup vs baseline: 8.0599x; 1.0111x over previous
import functools

import jax
import jax.numpy as jnp
from jax import lax
from jax.experimental import pallas as pl
from jax.experimental.pallas import tpu as pltpu
from jax.experimental.pallas import tpu_sc as plsc

CASCADE = 4
GRID = 128
M = GRID ** 3
N = M // 4 * 2
DECAY = 0.95
DENSITY_SCALE = 1.0

NC = 2
NS = 16
SLICE = M // (NC * NS)
NR = 4
QUART = N // NR
EPT = QUART // NS

CHUNK = 2048
NGRP = EPT // CHUNK
DWIN = 4096

CAPE = QUART + 15 * 256
CAPS = CAPE + NS * 16
SENTINEL = 0x7FFFFFF0


def _iota16():
    return lax.iota(jnp.int32, 16)


_MESH = dict(
    mesh=plsc.VectorSubcoreMesh(core_axis_name="c", subcore_axis_name="s"),
    compiler_params=pltpu.CompilerParams(needs_layout_passes=False),
)


def _k1_body(idx_hbm, cnt_hbm, cnt_v, ib):
    k = lax.axis_index("c")
    s = lax.axis_index("s")
    w2 = k * NS + s
    for u in range(16):
        cnt_v[pl.ds(u * 16, 16)] = jnp.zeros((16,), jnp.int32)
    for c in range(CASCADE):
        for r in range(NR):
            elem0 = c * N + r * QUART + s * EPT

            def _win(w, carry):
                pltpu.sync_copy(idx_hbm.at[pl.ds(elem0 + w * DWIN, DWIN)], ib)

                def _vec(v, c2):
                    for u in range(4):
                        iv = ib[pl.ds((v * 4 + u) * 16, 16)]
                        o = lax.shift_right_logical(iv, 16)
                        keep = lax.shift_right_logical(o, 4) == k
                        o16 = o & 15
                        cnt, last = plsc.scan_count(o16, mask=keep)
                        plsc.addupdate_scatter(
                            cnt_v, [o16 + (c * 64 + r * 16)], cnt, mask=last)
                    return c2
                lax.fori_loop(0, DWIN // 64, _vec, 0)
                return carry
            lax.fori_loop(0, EPT // DWIN, _win, 0)
    pltpu.sync_copy(cnt_v, cnt_hbm.at[pl.ds(w2 * 256, 256)])


def _k2_body(cnt_hbm, offs_hbm, segb_hbm, cnt4k, offs4k, segb_v):
    k = lax.axis_index("c")
    s = lax.axis_index("s")

    @pl.when((k == 0) & (s == 0))
    def _():
        pltpu.sync_copy(cnt_hbm, cnt4k)
        ii = _iota16()

        def _sb(p, carry):
            q = p * 16 + ii
            qk = lax.shift_right_logical(q, 12)
            qc = lax.shift_right_logical(q, 10) & 3
            qr = lax.shift_right_logical(q, 8) & 3
            qo = lax.shift_right_logical(q, 4) & 15
            qs = q & 15
            src = (qk * NS + qs) * 256 + qc * 64 + qr * 16 + qo
            v = plsc.load_gather(cnt4k, [src])
            pv = (v + 15) & jnp.int32(-16)
            inc = plsc.cumsum(pv)
            ex = inc - pv + carry
            plsc.store_scatter(offs4k, [src], ex)
            grp = lax.shift_right_logical(p, 4)
            plsc.store_scatter(segb_v, [grp * 32 + qo], ex, mask=(qs == 0))
            carry_new = carry + jnp.max(inc)
            gpos = p & 15
            plsc.store_scatter(
                segb_v, [jnp.full((16,), 1, jnp.int32) * (grp * 32 + 16)],
                jnp.full((16,), 1, jnp.int32) * carry_new,
                mask=(ii == 0) & (gpos == 15))
            return jnp.where(gpos == 15, jnp.int32(0), carry_new)
        lax.fori_loop(0, 512, _sb, jnp.int32(0))
        pltpu.sync_copy(offs4k, offs_hbm)
        pltpu.sync_copy(segb_v, segb_hbm)


def _k34_body(idx_hbm, sig_hbm, offs_hbm, segb_hbm, tmp_hbm,
              offs_v, segb_v, cur_v, ib, sb,
              dbuf0, dbuf1, vbuf0, vbuf1, wbuf0, wbuf1,
              dpad, vpad, wpad, ib4, sb4, tmp_v,
              sidx_sp, ssig_sp, sem_i, sem_s):
    dbufs, vbufs, wbufs = (dbuf0, dbuf1), (vbuf0, vbuf1), (wbuf0, wbuf1)
    k = lax.axis_index("c")
    s = lax.axis_index("s")
    w2 = k * NS + s
    ii = _iota16()
    trash = jnp.int32(CAPE) + s * 16
    pltpu.sync_copy(offs_hbm.at[pl.ds(w2 * 256, 256)], offs_v)
    pltpu.sync_copy(segb_hbm, segb_v)
    own_base = (k * NS + s) * SLICE

    for c in range(CASCADE):
        for r in range(NR):
            cur_v[pl.ds(0, 16)] = offs_v[pl.ds(c * 64 + r * 16, 16)]
            elem0 = c * N + r * QUART + s * EPT

            def _group(g, ph):
                pltpu.sync_copy(idx_hbm.at[pl.ds(elem0 + g * CHUNK, CHUNK)],
                                ib)
                pltpu.sync_copy(sig_hbm.at[pl.ds(elem0 + g * CHUNK, CHUNK)],
                                sb)

                @pl.when(g >= 2)
                def _():
                    pltpu.make_async_copy(
                        vbufs[ph], sidx_sp.at[pl.ds(0, CHUNK)],
                        sem_i.at[ph]).wait()
                    pltpu.make_async_copy(
                        wbufs[ph], ssig_sp.at[pl.ds(0, CHUNK)],
                        sem_s.at[ph]).wait()

                def _vec(v, c2):
                    for u in range(4):
                        iv = ib[pl.ds((v * 4 + u) * 16, 16)]
                        sv = sb[pl.ds((v * 4 + u) * 16, 16)]
                        o = lax.shift_right_logical(iv, 16)
                        keep = lax.shift_right_logical(o, 4) == k
                        o16 = o & 15
                        cnt, last = plsc.scan_count(o16, mask=keep)
                        base = plsc.load_gather(cur_v, [o16])
                        dest = base + cnt - 1
                        plsc.store_scatter(cur_v, [o16], dest + 1, mask=last)
                        dbufs[ph][pl.ds((v * 4 + u) * 16, 16)] = jnp.where(
                            keep, dest, trash + ii)
                        vbufs[ph][pl.ds((v * 4 + u) * 16, 16)] = iv
                        wbufs[ph][pl.ds((v * 4 + u) * 16, 16)] = sv
                    return c2
                lax.fori_loop(0, CHUNK // 64, _vec, 0)
                pltpu.async_copy(vbufs[ph], sidx_sp.at[dbufs[ph]],
                                 sem_i.at[ph])
                pltpu.async_copy(wbufs[ph], ssig_sp.at[dbufs[ph]],
                                 sem_s.at[ph])

            def _gpair(gp, carry):
                _group(gp * 2, 0)
                _group(gp * 2 + 1, 1)
                return carry
            lax.fori_loop(0, NGRP // 2, _gpair, 0)
            for ph in range(2):
                pltpu.make_async_copy(vbufs[ph], sidx_sp.at[pl.ds(0, CHUNK)],
                                      sem_i.at[ph]).wait()
                pltpu.make_async_copy(wbufs[ph], ssig_sp.at[pl.ds(0, CHUNK)],
                                      sem_s.at[ph]).wait()

            cv0 = cur_v[pl.ds(0, 16)]
            for o in range(NS):
                cur_o = cv0[o]
                npad = (-cur_o) & 15
                dd = jnp.where(ii < npad, cur_o + ii, trash + ii)
                dpad[pl.ds(o * 16, 16)] = dd
                vpad[pl.ds(o * 16, 16)] = jnp.full((16,), SENTINEL, jnp.int32)
                wpad[pl.ds(o * 16, 16)] = jnp.zeros((16,), jnp.float32)
            pltpu.sync_copy(vpad, sidx_sp.at[dpad])
            pltpu.sync_copy(wpad, ssig_sp.at[dpad])

            plsc.subcore_barrier()

            if r == 0:
                def _ms(i, carry):
                    for u in range(8):
                        tmp_v[pl.ds((i * 8 + u) * 16, 16)] = jnp.full(
                            (16,), -1.0, jnp.float32)
                    return carry
                lax.fori_loop(0, SLICE // 128, _ms, 0)

            grp = k * 16 + c * NR + r
            se = plsc.load_gather(segb_v, [grp * 32 + s + ii])
            start = se[0]
            end0 = se[1]
            nwin = lax.shift_right_logical(end0 - start + (DWIN - 1), 12)

            def _win(w, carry):
                off_l = start + w * DWIN
                off = pl.multiple_of(
                    jnp.maximum(jnp.minimum(off_l, jnp.int32(CAPE - DWIN)),
                                jnp.int32(0)), 16)
                pltpu.sync_copy(sidx_sp.at[pl.ds(off, DWIN)], ib4)
                pltpu.sync_copy(ssig_sp.at[pl.ds(off, DWIN)], sb4)

                def _vec(v, c2):
                    for u in range(4):
                        iv = ib4[pl.ds((v * 4 + u) * 16, 16)]
                        sv = sb4[pl.ds((v * 4 + u) * 16, 16)]
                        gpos = off + (v * 4 + u) * 16 + ii
                        loc = iv - own_base
                        ok = ((gpos >= off_l) & (gpos < end0)
                              & (loc >= 0) & (loc < SLICE))
                        locc = jnp.where(ok, loc, 0)
                        _, last = plsc.scan_count(locc, mask=ok)
                        plsc.store_scatter(tmp_v, [locc], sv, mask=last)
                    return c2
                lax.fori_loop(0, DWIN // 64, _vec, 0)
                return carry
            lax.fori_loop(0, nwin, _win, 0)

            if r == NR - 1:
                pltpu.sync_copy(
                    tmp_v, tmp_hbm.at[pl.ds(c * M + own_base, SLICE)])

            plsc.subcore_barrier()


@jax.jit
def _sc_scatter(idx_flat, sig_flat):
    i32 = jnp.int32
    f32 = jnp.float32
    k1 = functools.partial(
        pl.kernel,
        out_type=jax.ShapeDtypeStruct((NC * NS * 256,), i32),
        scratch_types=[
            pltpu.VMEM((256,), i32),
            pltpu.VMEM((DWIN,), i32),
        ],
        **_MESH,
    )(_k1_body)
    cnts = k1(idx_flat)

    k2 = functools.partial(
        pl.kernel,
        out_type=(jax.ShapeDtypeStruct((NC * NS * 256,), i32),
                  jax.ShapeDtypeStruct((1024,), i32)),
        scratch_types=[
            pltpu.VMEM((NC * NS * 256,), i32),
            pltpu.VMEM((NC * NS * 256,), i32),
            pltpu.VMEM((1024,), i32),
        ],
        **_MESH,
    )(_k2_body)
    offs, segb = k2(cnts)

    k34 = functools.partial(
        pl.kernel,
        out_type=jax.ShapeDtypeStruct((CASCADE * M,), f32),
        scratch_types=[
            pltpu.VMEM((256,), i32),
            pltpu.VMEM((1024,), i32),
            pltpu.VMEM((16,), i32),
            pltpu.VMEM((CHUNK,), i32),
            pltpu.VMEM((CHUNK,), f32),
            pltpu.VMEM((CHUNK,), i32),
            pltpu.VMEM((CHUNK,), i32),
            pltpu.VMEM((CHUNK,), i32),
            pltpu.VMEM((CHUNK,), i32),
            pltpu.VMEM((CHUNK,), f32),
            pltpu.VMEM((CHUNK,), f32),
            pltpu.VMEM((NS * 16,), i32),
            pltpu.VMEM((NS * 16,), i32),
            pltpu.VMEM((NS * 16,), f32),
            pltpu.VMEM((DWIN,), i32),
            pltpu.VMEM((DWIN,), f32),
            pltpu.VMEM((SLICE,), f32),
            pltpu.VMEM_SHARED((CAPS,), i32),
            pltpu.VMEM_SHARED((CAPS,), f32),
            pltpu.SemaphoreType.DMA((2,)),
            pltpu.SemaphoreType.DMA((2,)),
        ],
        **_MESH,
    )(_k34_body)
    return k34(idx_flat, sig_flat, offs, segb)


def _tc_body(g_ref, t_ref, o_ref):
    g = g_ref[...]
    t = t_ref[...]
    o_ref[...] = jnp.where((g >= 0.0) & (t >= 0.0),
                           jnp.maximum(g * DECAY, t), g)


@jax.jit
def _tc_combine(density_grid, tmp_grid):
    C, Mc = density_grid.shape
    rows, cols = 8192, C * Mc // 8192
    blk = 512
    out = pl.pallas_call(
        _tc_body,
        out_shape=jax.ShapeDtypeStruct((rows, cols), jnp.float32),
        grid=(rows // blk,),
        in_specs=[
            pl.BlockSpec((blk, cols), lambda i: (i, 0)),
            pl.BlockSpec((blk, cols), lambda i: (i, 0)),
        ],
        out_specs=pl.BlockSpec((blk, cols), lambda i: (i, 0)),
    )(density_grid.reshape(rows, cols), tmp_grid.reshape(rows, cols))
    return out.reshape(C, Mc)


def kernel(density_grid, sigmas, indices):
    C, Mc = density_grid.shape
    idx_flat = indices.reshape(-1)
    sig_flat = (sigmas * DENSITY_SCALE).reshape(-1)
    tmp = _sc_scatter(idx_flat, sig_flat).reshape(C, Mc)
    return _tc_combine(density_grid, tmp)
